# baseline (speedup 1.0000x reference)
import functools

import jax
import jax.numpy as jnp
from jax import lax
from jax.experimental import pallas as pl
from jax.experimental.pallas import tpu as pltpu
from jax.experimental.pallas import tpu_sc as plsc

N = 50000
E = 800000
D = 128
H = 64
HH = 32
OUT = 2

CHUNK = 128
NROWS = E // CHUNK
AGRP = 3
CGRP = 6
NROWS_PAD = 6336
NSUB = 16
NCORE = 2
ROWS_PER_TILE = 3128
N_PAD = ROWS_PER_TILE * NSUB
SPAN = 3200
CNT_PAD = SPAN * NSUB

_mesh = plsc.VectorSubcoreMesh(core_axis_name="c", subcore_axis_name="s")
_SC_PARAMS = pltpu.CompilerParams(use_tc_tiling_on_sc=False)


_CW = 16
_CNG = NROWS_PAD // (NCORE * NSUB) // CGRP


@functools.partial(
    pl.kernel,
    out_type=[jax.ShapeDtypeStruct((NCORE, CNT_PAD, _CW), jnp.float32),
              jax.ShapeDtypeStruct((NCORE, CNT_PAD, _CW), jnp.float32)],
    mesh=_mesh,
    scratch_types=[
        pltpu.VMEM((2, CGRP, CHUNK), jnp.int32),
        pltpu.VMEM((CHUNK, _CW), jnp.float32),
        pltpu.VMEM_SHARED((CNT_PAD, _CW), jnp.float32),
        pltpu.SemaphoreType.DMA,
    ],
    compiler_params=_SC_PARAMS,
)
def _count_kernel(dst_a_hbm, dst_b_hbm, ones_hbm, zcol_hbm, out_a_hbm, out_b_hbm,
                  didx_v, ones_v, cnt_sh, sem):
    c = lax.axis_index("c")
    s = lax.axis_index("s")
    wid = c * NSUB + s
    stride = NCORE * NSUB
    pltpu.sync_copy(ones_hbm, ones_v)

    def one_pass(dst_hbm, out_hbm):
        pltpu.sync_copy(zcol_hbm, cnt_sh.at[pl.ds(s * SPAN, SPAN)])
        plsc.subcore_barrier()

        def fire_idx(g, p):
            for i in range(CGRP):
                pltpu.async_copy(dst_hbm.at[wid + (g * CGRP + i) * stride],
                                 didx_v.at[p, i], sem)

        def drain_idx(p):
            for i in range(CGRP):
                pltpu.make_async_copy(dst_hbm.at[0], didx_v.at[p, i], sem).wait()

        fire_idx(0, 0)

        def body(g, carry):
            p = lax.rem(g, 2)
            drain_idx(p)

            @pl.when(g + 1 < _CNG)
            def _():
                fire_idx(g + 1, 1 - p)

            for i in range(CGRP):
                pltpu.sync_copy(ones_v, cnt_sh.at[didx_v.at[p, i]], add=True)
            return carry

        lax.fori_loop(0, _CNG, body, 0)
        plsc.subcore_barrier()
        pltpu.sync_copy(
            cnt_sh.at[pl.ds(s * SPAN, SPAN)],
            out_hbm.at[c].at[pl.ds(s * SPAN, SPAN)],
        )

    one_pass(dst_a_hbm, out_a_hbm)
    one_pass(dst_b_hbm, out_b_hbm)


def _counts(dst_a, dst_b):
    ones = jnp.ones((CHUNK, _CW), jnp.float32)
    zcol = jnp.zeros((SPAN, _CW), jnp.float32)
    ca, cb = _count_kernel(dst_a, dst_b, ones, zcol)
    return ca[:, :, :1], cb[:, :, :1]


_RING = 4
_ANC = NROWS_PAD // NSUB


@functools.partial(
    pl.kernel,
    out_type=jax.ShapeDtypeStruct((NCORE, N_PAD, HH), jnp.float32),
    mesh=_mesh,
    scratch_types=[
        pltpu.VMEM((_RING, CHUNK), jnp.int32),
        pltpu.VMEM((_RING, CHUNK), jnp.int32),
        pltpu.VMEM((_RING, CHUNK, HH), jnp.float32),
        pltpu.VMEM_SHARED((N_PAD, HH), jnp.float32),
        pltpu.SemaphoreType.DMA,
        pltpu.SemaphoreType.DMA,
        pltpu.SemaphoreType.DMA,
    ],
    compiler_params=_SC_PARAMS,
)
def _agg_kernel(tab0, tab1, src2d, dst2d, zrows_hbm, out_hbm,
                sidx_v, didx_v, rows_v, agg_sh, sem_i, sem_g, sem_s):
    c = lax.axis_index("c")
    s = lax.axis_index("s")
    base = s * ROWS_PER_TILE

    def fire_idx(g, b):
        j = s + g * NSUB
        pltpu.async_copy(src2d.at[j], sidx_v.at[b], sem_i)
        pltpu.async_copy(dst2d.at[j], didx_v.at[b], sem_i)

    def drain_idx(b):
        pltpu.make_async_copy(src2d.at[0], sidx_v.at[b], sem_i).wait()
        pltpu.make_async_copy(dst2d.at[0], didx_v.at[b], sem_i).wait()

    def fire_gather(b):
        @pl.when(c == 0)
        def _():
            pltpu.async_copy(tab0.at[sidx_v.at[b]], rows_v.at[b], sem_g)

        @pl.when(c == 1)
        def _():
            pltpu.async_copy(tab1.at[sidx_v.at[b]], rows_v.at[b], sem_g)

    def drain_gather(b):
        pltpu.make_async_copy(tab0.at[sidx_v.at[b]], rows_v.at[b], sem_g).wait()

    def fire_scatter(b):
        pltpu.async_copy(rows_v.at[b], agg_sh.at[didx_v.at[b]], sem_s, add=True)

    def drain_scatter(b):
        pltpu.make_async_copy(rows_v.at[b], agg_sh.at[didx_v.at[b]],
                              sem_s).wait()

    pltpu.sync_copy(zrows_hbm, agg_sh.at[pl.ds(base, ROWS_PER_TILE)])
    plsc.subcore_barrier()

    fire_idx(0, 0)
    fire_idx(1, 1)
    drain_idx(0)
    fire_gather(0)

    def body(g, carry):
        b = lax.rem(g, _RING)
        b1 = lax.rem(g + 1, _RING)
        b2 = lax.rem(g + 2, _RING)
        drain_gather(b)

        @pl.when(g + 1 < _ANC)
        def _():
            drain_idx(b1)
            fire_gather(b1)

        fire_scatter(b)

        @pl.when(g >= 2)
        def _():
            drain_scatter(b2)

        @pl.when(g + 2 < _ANC)
        def _():
            fire_idx(g + 2, b2)

        return carry

    lax.fori_loop(0, _ANC, body, 0)
    for t in (_ANC - 2, _ANC - 1):
        drain_scatter(t % _RING)
    plsc.subcore_barrier()
    pltpu.sync_copy(
        agg_sh.at[pl.ds(base, ROWS_PER_TILE)],
        out_hbm.at[c].at[pl.ds(base, ROWS_PER_TILE)],
    )


def _aggregate(table, src2d, dst2d):
    zrows = jnp.zeros((ROWS_PER_TILE, HH), jnp.float32)
    return _agg_kernel(table[0], table[1], src2d, dst2d, zrows)


_RB = 2000
_GRID = N // _RB


def _proj_body(x_ref, w_ref, b_ref, out_ref):
    y = jnp.dot(x_ref[...], w_ref[...], preferred_element_type=jnp.float32)
    y = y + b_ref[...]
    out_ref[0] = y[:, :HH]
    out_ref[1] = y[:, HH:]


def _project(x, W, b):
    return pl.pallas_call(
        _proj_body,
        grid=(_GRID,),
        in_specs=[
            pl.BlockSpec((_RB, D), lambda i: (i, 0)),
            pl.BlockSpec((D, H), lambda i: (0, 0)),
            pl.BlockSpec((1, H), lambda i: (0, 0)),
        ],
        out_specs=pl.BlockSpec((NCORE, _RB, HH), lambda i: (0, i, 0)),
        out_shape=jax.ShapeDtypeStruct((NCORE, N, HH), jnp.float32),
    )(x, W, b.reshape(1, H))


def _sage_mix(agg_ref, cnt_ref, xd_ref, wl_ref, bl_ref, wr_ref):
    cnt = jnp.maximum(cnt_ref[0] + cnt_ref[1], 1.0)
    mean = jnp.concatenate([agg_ref[0], agg_ref[1]], axis=1) / cnt
    xd = jnp.concatenate([xd_ref[0], xd_ref[1]], axis=1)
    y = (
        jnp.dot(mean, wl_ref[...], preferred_element_type=jnp.float32)
        + bl_ref[...]
        + jnp.dot(xd, wr_ref[...], preferred_element_type=jnp.float32)
    )
    return jnp.maximum(y, 0.0)


def _mid_body(agg_ref, cnt_ref, xd_ref, wl_ref, bl_ref, wr_ref, out_ref):
    y = _sage_mix(agg_ref, cnt_ref, xd_ref, wl_ref, bl_ref, wr_ref)
    out_ref[0] = y[:, :HH]
    out_ref[1] = y[:, HH:]


def _final_body(agg_ref, cnt_ref, xd_ref, wl_ref, bl_ref, wr_ref, out_ref):
    out_ref[...] = _sage_mix(agg_ref, cnt_ref, xd_ref, wl_ref, bl_ref, wr_ref)


def _head_body(agg_ref, cnt_ref, xd_ref, wl_ref, bl_ref, wr_ref,
               w1_ref, b1_ref, w2_ref, b2_ref, out_ref, logits_ref):
    y = _sage_mix(agg_ref, cnt_ref, xd_ref, wl_ref, bl_ref, wr_ref)
    out_ref[...] = y
    h = jnp.maximum(
        jnp.dot(y, w1_ref[...], preferred_element_type=jnp.float32) + b1_ref[...],
        0.0,
    )
    logits_ref[...] = (
        jnp.dot(h, w2_ref[...], preferred_element_type=jnp.float32) + b2_ref[...]
    )


_SAGE_SPECS = [
    pl.BlockSpec((NCORE, _RB, HH), lambda i: (0, i, 0)),
    pl.BlockSpec((NCORE, _RB, 1), lambda i: (0, i, 0)),
    pl.BlockSpec((NCORE, _RB, HH), lambda i: (0, i, 0)),
    pl.BlockSpec((H, H), lambda i: (0, 0)),
    pl.BlockSpec((1, H), lambda i: (0, 0)),
    pl.BlockSpec((H, H), lambda i: (0, 0)),
]


def _sage_mid(agg, cnt, xd, Wl, bl, Wr):
    return pl.pallas_call(
        _mid_body,
        grid=(_GRID,),
        in_specs=_SAGE_SPECS,
        out_specs=pl.BlockSpec((NCORE, _RB, HH), lambda i: (0, i, 0)),
        out_shape=jax.ShapeDtypeStruct((NCORE, N, HH), jnp.float32),
    )(agg, cnt, xd, Wl, bl.reshape(1, H), Wr)


def _sage_final(agg, cnt, xd, Wl, bl, Wr):
    return pl.pallas_call(
        _final_body,
        grid=(_GRID,),
        in_specs=_SAGE_SPECS,
        out_specs=pl.BlockSpec((_RB, H), lambda i: (i, 0)),
        out_shape=jax.ShapeDtypeStruct((N, H), jnp.float32),
    )(agg, cnt, xd, Wl, bl.reshape(1, H), Wr)


def _sage_head(agg, cnt, xd, Wl, bl, Wr, W1, b1, W2, b2):
    return pl.pallas_call(
        _head_body,
        grid=(_GRID,),
        in_specs=_SAGE_SPECS + [
            pl.BlockSpec((H, H), lambda i: (0, 0)),
            pl.BlockSpec((1, H), lambda i: (0, 0)),
            pl.BlockSpec((H, OUT), lambda i: (0, 0)),
            pl.BlockSpec((1, OUT), lambda i: (0, 0)),
        ],
        out_specs=[
            pl.BlockSpec((_RB, H), lambda i: (i, 0)),
            pl.BlockSpec((_RB, OUT), lambda i: (i, 0)),
        ],
        out_shape=[
            jax.ShapeDtypeStruct((N, H), jnp.float32),
            jax.ShapeDtypeStruct((N, OUT), jnp.float32),
        ],
    )(agg, cnt, xd, Wl, bl.reshape(1, H), Wr,
      W1, b1.reshape(1, H), W2, b2.reshape(1, OUT))


def kernel(x_transaction, x_account, edge_index_t2a, edge_index_a2t,
           W_in_t, b_in_t, W_in_a, b_in_a,
           c1_t2a_Wl, c1_t2a_bl, c1_t2a_Wr,
           c1_a2t_Wl, c1_a2t_bl, c1_a2t_Wr,
           c2_t2a_Wl, c2_t2a_bl, c2_t2a_Wr,
           c2_a2t_Wl, c2_a2t_bl, c2_a2t_Wr,
           head_W1, head_b1, head_W2, head_b2):
    pad_rows = NROWS_PAD - NROWS
    src_pad = jnp.zeros((pad_rows, CHUNK), jnp.int32)
    dst_pad = jnp.full((pad_rows, CHUNK), N, jnp.int32)

    def _edges(ei):
        e32 = ei.astype(jnp.int32)
        src = jnp.concatenate([e32[0].reshape(NROWS, CHUNK), src_pad])
        dst = jnp.concatenate([e32[1].reshape(NROWS, CHUNK), dst_pad])
        return src, dst

    src_t2a, dst_t2a = _edges(edge_index_t2a)
    src_a2t, dst_a2t = _edges(edge_index_a2t)

    cnt_t2a, cnt_a2t = _counts(dst_t2a, dst_a2t)

    t_tab = _project(x_transaction, W_in_t, b_in_t)
    a_tab = _project(x_account, W_in_a, b_in_a)

    agg_a = _aggregate(t_tab, src_t2a, dst_t2a)
    agg_t = _aggregate(a_tab, src_a2t, dst_a2t)
    xa1 = _sage_mid(agg_a, cnt_t2a, a_tab, c1_t2a_Wl, c1_t2a_bl, c1_t2a_Wr)
    xt1 = _sage_mid(agg_t, cnt_a2t, t_tab, c1_a2t_Wl, c1_a2t_bl, c1_a2t_Wr)

    agg_a2 = _aggregate(xt1, src_t2a, dst_t2a)
    agg_t2 = _aggregate(xa1, src_a2t, dst_a2t)
    xa2 = _sage_final(agg_a2, cnt_t2a, xa1, c2_t2a_Wl, c2_t2a_bl, c2_t2a_Wr)
    xt2, logits = _sage_head(agg_t2, cnt_a2t, xt1,
                             c2_a2t_Wl, c2_a2t_bl, c2_a2t_Wr,
                             head_W1, head_b1, head_W2, head_b2)
    return (logits, xt2, xa2)


# speedup vs baseline: 1.1916x; 1.1916x over previous
import functools

import jax
import jax.numpy as jnp
from jax import lax
from jax.experimental import pallas as pl
from jax.experimental.pallas import tpu as pltpu
from jax.experimental.pallas import tpu_sc as plsc

N = 50000
E = 800000
D = 128
H = 64
HH = 32
OUT = 2

CHUNK = 128
NROWS = E // CHUNK
AGRP = 3
CGRP = 6
NROWS_PAD = 6336
NSUB = 16
NCORE = 2
ROWS_PER_TILE = 3128
N_PAD = ROWS_PER_TILE * NSUB
SPAN = 3200
CNT_PAD = SPAN * NSUB

_mesh = plsc.VectorSubcoreMesh(core_axis_name="c", subcore_axis_name="s")
_SC_PARAMS = pltpu.CompilerParams(use_tc_tiling_on_sc=False)


_CW = 16
_CNG = NROWS_PAD // (NCORE * NSUB) // CGRP


@functools.partial(
    pl.kernel,
    out_type=[jax.ShapeDtypeStruct((NCORE, CNT_PAD, _CW), jnp.float32),
              jax.ShapeDtypeStruct((NCORE, CNT_PAD, _CW), jnp.float32)],
    mesh=_mesh,
    scratch_types=[
        pltpu.VMEM((2, CGRP, CHUNK), jnp.int32),
        pltpu.VMEM((CHUNK, _CW), jnp.float32),
        pltpu.VMEM_SHARED((CNT_PAD, _CW), jnp.float32),
        pltpu.SemaphoreType.DMA,
    ],
    compiler_params=_SC_PARAMS,
)
def _count_kernel(dst_a_hbm, dst_b_hbm, ones_hbm, zcol_hbm, out_a_hbm, out_b_hbm,
                  didx_v, ones_v, cnt_sh, sem):
    c = lax.axis_index("c")
    s = lax.axis_index("s")
    wid = c * NSUB + s
    stride = NCORE * NSUB
    pltpu.sync_copy(ones_hbm, ones_v)

    def one_pass(dst_hbm, out_hbm):
        pltpu.sync_copy(zcol_hbm, cnt_sh.at[pl.ds(s * SPAN, SPAN)])
        plsc.subcore_barrier()

        def fire_idx(g, p):
            for i in range(CGRP):
                pltpu.async_copy(dst_hbm.at[wid + (g * CGRP + i) * stride],
                                 didx_v.at[p, i], sem)

        def drain_idx(p):
            for i in range(CGRP):
                pltpu.make_async_copy(dst_hbm.at[0], didx_v.at[p, i], sem).wait()

        fire_idx(0, 0)

        def body(g, carry):
            p = lax.rem(g, 2)
            drain_idx(p)

            @pl.when(g + 1 < _CNG)
            def _():
                fire_idx(g + 1, 1 - p)

            for i in range(CGRP):
                pltpu.sync_copy(ones_v, cnt_sh.at[didx_v.at[p, i]], add=True)
            return carry

        lax.fori_loop(0, _CNG, body, 0)
        plsc.subcore_barrier()
        pltpu.sync_copy(
            cnt_sh.at[pl.ds(s * SPAN, SPAN)],
            out_hbm.at[c].at[pl.ds(s * SPAN, SPAN)],
        )

    one_pass(dst_a_hbm, out_a_hbm)
    one_pass(dst_b_hbm, out_b_hbm)


def _counts(dst_a, dst_b):
    ones = jnp.ones((CHUNK, _CW), jnp.float32)
    zcol = jnp.zeros((SPAN, _CW), jnp.float32)
    ca, cb = _count_kernel(dst_a, dst_b, ones, zcol)
    return ca[:, :, :1], cb[:, :, :1]


_RING = 6
_GLA = 3
_ILA = 4
_ANC = NROWS_PAD // NSUB


@functools.partial(
    pl.kernel,
    out_type=jax.ShapeDtypeStruct((NCORE, N_PAD, HH), jnp.float32),
    mesh=_mesh,
    scratch_types=[
        pltpu.VMEM((_RING, CHUNK), jnp.int32),
        pltpu.VMEM((_RING, CHUNK), jnp.int32),
        pltpu.VMEM((_RING, CHUNK, HH), jnp.float32),
        pltpu.VMEM_SHARED((N_PAD, HH), jnp.float32),
        pltpu.SemaphoreType.DMA,
        pltpu.SemaphoreType.DMA,
        pltpu.SemaphoreType.DMA,
    ],
    compiler_params=_SC_PARAMS,
)
def _agg_kernel(tab0, tab1, src2d, dst2d, zrows_hbm, out_hbm,
                sidx_v, didx_v, rows_v, agg_sh, sem_i, sem_g, sem_s):
    c = lax.axis_index("c")
    s = lax.axis_index("s")
    base = s * ROWS_PER_TILE

    def fire_idx(g, b):
        j = s + g * NSUB
        pltpu.async_copy(src2d.at[j], sidx_v.at[b], sem_i)
        pltpu.async_copy(dst2d.at[j], didx_v.at[b], sem_i)

    def drain_idx(b):
        pltpu.make_async_copy(src2d.at[0], sidx_v.at[b], sem_i).wait()
        pltpu.make_async_copy(dst2d.at[0], didx_v.at[b], sem_i).wait()

    def fire_gather(b):
        @pl.when(c == 0)
        def _():
            pltpu.async_copy(tab0.at[sidx_v.at[b]], rows_v.at[b], sem_g)

        @pl.when(c == 1)
        def _():
            pltpu.async_copy(tab1.at[sidx_v.at[b]], rows_v.at[b], sem_g)

    def drain_gather(b):
        pltpu.make_async_copy(tab0.at[sidx_v.at[b]], rows_v.at[b], sem_g).wait()

    def fire_scatter(b):
        pltpu.async_copy(rows_v.at[b], agg_sh.at[didx_v.at[b]], sem_s, add=True)

    def drain_scatter(b):
        pltpu.make_async_copy(rows_v.at[b], agg_sh.at[didx_v.at[b]],
                              sem_s).wait()

    pltpu.sync_copy(zrows_hbm, agg_sh.at[pl.ds(base, ROWS_PER_TILE)])
    plsc.subcore_barrier()

    for g in range(_ILA):
        fire_idx(g, g)
    for g in range(_GLA):
        drain_idx(g)
        fire_gather(g)

    def body(g, carry):
        b = lax.rem(g, _RING)
        bg = lax.rem(g + _GLA, _RING)
        bi = lax.rem(g + _ILA, _RING)
        drain_gather(b)

        @pl.when(g + _GLA < _ANC)
        def _():
            drain_idx(bg)
            fire_gather(bg)

        fire_scatter(b)

        @pl.when(g >= 2)
        def _():
            drain_scatter(bi)

        @pl.when(g + _ILA < _ANC)
        def _():
            fire_idx(g + _ILA, bi)

        return carry

    lax.fori_loop(0, _ANC, body, 0)
    for t in (_ANC - 2, _ANC - 1):
        drain_scatter(t % _RING)
    plsc.subcore_barrier()
    pltpu.sync_copy(
        agg_sh.at[pl.ds(base, ROWS_PER_TILE)],
        out_hbm.at[c].at[pl.ds(base, ROWS_PER_TILE)],
    )


def _aggregate(table, src2d, dst2d):
    zrows = jnp.zeros((ROWS_PER_TILE, HH), jnp.float32)
    return _agg_kernel(table[0], table[1], src2d, dst2d, zrows)


_RB = 2000
_GRID = N // _RB


def _proj_body(x_ref, w_ref, b_ref, out_ref):
    y = jnp.dot(x_ref[...], w_ref[...], preferred_element_type=jnp.float32)
    y = y + b_ref[...]
    out_ref[0] = y[:, :HH]
    out_ref[1] = y[:, HH:]


def _project(x, W, b):
    return pl.pallas_call(
        _proj_body,
        grid=(_GRID,),
        in_specs=[
            pl.BlockSpec((_RB, D), lambda i: (i, 0)),
            pl.BlockSpec((D, H), lambda i: (0, 0)),
            pl.BlockSpec((1, H), lambda i: (0, 0)),
        ],
        out_specs=pl.BlockSpec((NCORE, _RB, HH), lambda i: (0, i, 0)),
        out_shape=jax.ShapeDtypeStruct((NCORE, N, HH), jnp.float32),
    )(x, W, b.reshape(1, H))


def _sage_mix(agg_ref, cnt_ref, xd_ref, wl_ref, bl_ref, wr_ref):
    cnt = jnp.maximum(cnt_ref[0] + cnt_ref[1], 1.0)
    mean = jnp.concatenate([agg_ref[0], agg_ref[1]], axis=1) / cnt
    xd = jnp.concatenate([xd_ref[0], xd_ref[1]], axis=1)
    y = (
        jnp.dot(mean, wl_ref[...], preferred_element_type=jnp.float32)
        + bl_ref[...]
        + jnp.dot(xd, wr_ref[...], preferred_element_type=jnp.float32)
    )
    return jnp.maximum(y, 0.0)


def _mid_body(agg_ref, cnt_ref, xd_ref, wl_ref, bl_ref, wr_ref, out_ref):
    y = _sage_mix(agg_ref, cnt_ref, xd_ref, wl_ref, bl_ref, wr_ref)
    out_ref[0] = y[:, :HH]
    out_ref[1] = y[:, HH:]


def _final_body(agg_ref, cnt_ref, xd_ref, wl_ref, bl_ref, wr_ref, out_ref):
    out_ref[...] = _sage_mix(agg_ref, cnt_ref, xd_ref, wl_ref, bl_ref, wr_ref)


def _head_body(agg_ref, cnt_ref, xd_ref, wl_ref, bl_ref, wr_ref,
               w1_ref, b1_ref, w2_ref, b2_ref, out_ref, logits_ref):
    y = _sage_mix(agg_ref, cnt_ref, xd_ref, wl_ref, bl_ref, wr_ref)
    out_ref[...] = y
    h = jnp.maximum(
        jnp.dot(y, w1_ref[...], preferred_element_type=jnp.float32) + b1_ref[...],
        0.0,
    )
    logits_ref[...] = (
        jnp.dot(h, w2_ref[...], preferred_element_type=jnp.float32) + b2_ref[...]
    )


_SAGE_SPECS = [
    pl.BlockSpec((NCORE, _RB, HH), lambda i: (0, i, 0)),
    pl.BlockSpec((NCORE, _RB, 1), lambda i: (0, i, 0)),
    pl.BlockSpec((NCORE, _RB, HH), lambda i: (0, i, 0)),
    pl.BlockSpec((H, H), lambda i: (0, 0)),
    pl.BlockSpec((1, H), lambda i: (0, 0)),
    pl.BlockSpec((H, H), lambda i: (0, 0)),
]


def _sage_mid(agg, cnt, xd, Wl, bl, Wr):
    return pl.pallas_call(
        _mid_body,
        grid=(_GRID,),
        in_specs=_SAGE_SPECS,
        out_specs=pl.BlockSpec((NCORE, _RB, HH), lambda i: (0, i, 0)),
        out_shape=jax.ShapeDtypeStruct((NCORE, N, HH), jnp.float32),
    )(agg, cnt, xd, Wl, bl.reshape(1, H), Wr)


def _sage_final(agg, cnt, xd, Wl, bl, Wr):
    return pl.pallas_call(
        _final_body,
        grid=(_GRID,),
        in_specs=_SAGE_SPECS,
        out_specs=pl.BlockSpec((_RB, H), lambda i: (i, 0)),
        out_shape=jax.ShapeDtypeStruct((N, H), jnp.float32),
    )(agg, cnt, xd, Wl, bl.reshape(1, H), Wr)


def _sage_head(agg, cnt, xd, Wl, bl, Wr, W1, b1, W2, b2):
    return pl.pallas_call(
        _head_body,
        grid=(_GRID,),
        in_specs=_SAGE_SPECS + [
            pl.BlockSpec((H, H), lambda i: (0, 0)),
            pl.BlockSpec((1, H), lambda i: (0, 0)),
            pl.BlockSpec((H, OUT), lambda i: (0, 0)),
            pl.BlockSpec((1, OUT), lambda i: (0, 0)),
        ],
        out_specs=[
            pl.BlockSpec((_RB, H), lambda i: (i, 0)),
            pl.BlockSpec((_RB, OUT), lambda i: (i, 0)),
        ],
        out_shape=[
            jax.ShapeDtypeStruct((N, H), jnp.float32),
            jax.ShapeDtypeStruct((N, OUT), jnp.float32),
        ],
    )(agg, cnt, xd, Wl, bl.reshape(1, H), Wr,
      W1, b1.reshape(1, H), W2, b2.reshape(1, OUT))


def kernel(x_transaction, x_account, edge_index_t2a, edge_index_a2t,
           W_in_t, b_in_t, W_in_a, b_in_a,
           c1_t2a_Wl, c1_t2a_bl, c1_t2a_Wr,
           c1_a2t_Wl, c1_a2t_bl, c1_a2t_Wr,
           c2_t2a_Wl, c2_t2a_bl, c2_t2a_Wr,
           c2_a2t_Wl, c2_a2t_bl, c2_a2t_Wr,
           head_W1, head_b1, head_W2, head_b2):
    pad_rows = NROWS_PAD - NROWS
    src_pad = jnp.zeros((pad_rows, CHUNK), jnp.int32)
    dst_pad = jnp.full((pad_rows, CHUNK), N, jnp.int32)

    def _edges(ei):
        e32 = ei.astype(jnp.int32)
        src = jnp.concatenate([e32[0].reshape(NROWS, CHUNK), src_pad])
        dst = jnp.concatenate([e32[1].reshape(NROWS, CHUNK), dst_pad])
        return src, dst

    src_t2a, dst_t2a = _edges(edge_index_t2a)
    src_a2t, dst_a2t = _edges(edge_index_a2t)

    cnt_t2a, cnt_a2t = _counts(dst_t2a, dst_a2t)

    t_tab = _project(x_transaction, W_in_t, b_in_t)
    a_tab = _project(x_account, W_in_a, b_in_a)

    agg_a = _aggregate(t_tab, src_t2a, dst_t2a)
    agg_t = _aggregate(a_tab, src_a2t, dst_a2t)
    xa1 = _sage_mid(agg_a, cnt_t2a, a_tab, c1_t2a_Wl, c1_t2a_bl, c1_t2a_Wr)
    xt1 = _sage_mid(agg_t, cnt_a2t, t_tab, c1_a2t_Wl, c1_a2t_bl, c1_a2t_Wr)

    agg_a2 = _aggregate(xt1, src_t2a, dst_t2a)
    agg_t2 = _aggregate(xa1, src_a2t, dst_a2t)
    xa2 = _sage_final(agg_a2, cnt_t2a, xa1, c2_t2a_Wl, c2_t2a_bl, c2_t2a_Wr)
    xt2, logits = _sage_head(agg_t2, cnt_a2t, xt1,
                             c2_a2t_Wl, c2_a2t_bl, c2_a2t_Wr,
                             head_W1, head_b1, head_W2, head_b2)
    return (logits, xt2, xa2)


# speedup vs baseline: 1.3684x; 1.1483x over previous
import functools

import jax
import jax.numpy as jnp
from jax import lax
from jax.experimental import pallas as pl
from jax.experimental.pallas import tpu as pltpu
from jax.experimental.pallas import tpu_sc as plsc

N = 50000
E = 800000
D = 128
H = 64
HH = 32
OUT = 2

CHUNK = 128
NROWS = E // CHUNK
AGRP = 3
CGRP = 6
NROWS_PAD = 6336
NSUB = 16
NCORE = 2
ROWS_PER_TILE = 3128
N_PAD = ROWS_PER_TILE * NSUB
SPAN = 3200
CNT_PAD = SPAN * NSUB

_mesh = plsc.VectorSubcoreMesh(core_axis_name="c", subcore_axis_name="s")
_SC_PARAMS = pltpu.CompilerParams(use_tc_tiling_on_sc=False)


_CW = 16
_CNG = NROWS_PAD // (NCORE * NSUB) // CGRP


@functools.partial(
    pl.kernel,
    out_type=[jax.ShapeDtypeStruct((NCORE, CNT_PAD, _CW), jnp.float32),
              jax.ShapeDtypeStruct((NCORE, CNT_PAD, _CW), jnp.float32)],
    mesh=_mesh,
    scratch_types=[
        pltpu.VMEM((2, CGRP, CHUNK), jnp.int32),
        pltpu.VMEM((CHUNK, _CW), jnp.float32),
        pltpu.VMEM_SHARED((CNT_PAD, _CW), jnp.float32),
        pltpu.SemaphoreType.DMA,
    ],
    compiler_params=_SC_PARAMS,
)
def _count_kernel(ei_a_hbm, ei_b_hbm, ones_hbm, zcol_hbm, out_a_hbm, out_b_hbm,
                  didx_v, ones_v, cnt_sh, sem):
    c = lax.axis_index("c")
    s = lax.axis_index("s")
    wid = c * NSUB + s
    stride = NCORE * NSUB
    pltpu.sync_copy(ones_hbm, ones_v)

    def one_pass(ei_hbm, out_hbm):
        pltpu.sync_copy(zcol_hbm, cnt_sh.at[pl.ds(s * SPAN, SPAN)])
        plsc.subcore_barrier()

        def fire_idx(g, p):
            for i in range(CGRP):
                pltpu.async_copy(ei_hbm.at[wid + (g * CGRP + i) * stride, 1],
                                 didx_v.at[p, i], sem)

        def drain_idx(p):
            for i in range(CGRP):
                pltpu.make_async_copy(ei_hbm.at[0, 1], didx_v.at[p, i],
                                      sem).wait()

        fire_idx(0, 0)

        def body(g, carry):
            p = lax.rem(g, 2)
            drain_idx(p)

            @pl.when(g + 1 < _CNG)
            def _():
                fire_idx(g + 1, 1 - p)

            for i in range(CGRP):
                pltpu.sync_copy(ones_v, cnt_sh.at[didx_v.at[p, i]], add=True)
            return carry

        lax.fori_loop(0, _CNG, body, 0)
        plsc.subcore_barrier()
        pltpu.sync_copy(
            cnt_sh.at[pl.ds(s * SPAN, SPAN)],
            out_hbm.at[c].at[pl.ds(s * SPAN, SPAN)],
        )

    one_pass(ei_a_hbm, out_a_hbm)
    one_pass(ei_b_hbm, out_b_hbm)


def _counts(ei_a, ei_b):
    ones = jnp.ones((CHUNK, _CW), jnp.float32)
    zcol = jnp.zeros((SPAN, _CW), jnp.float32)
    ca, cb = _count_kernel(ei_a, ei_b, ones, zcol)
    return ca[:, :, :1], cb[:, :, :1]


_ANG = NROWS_PAD // NSUB // AGRP


@functools.partial(
    pl.kernel,
    out_type=jax.ShapeDtypeStruct((NCORE, N_PAD, HH), jnp.float32),
    mesh=_mesh,
    scratch_types=[
        pltpu.VMEM((2, AGRP, 2, CHUNK), jnp.int32),
        pltpu.VMEM((2, AGRP, CHUNK, HH), jnp.float32),
        pltpu.VMEM_SHARED((N_PAD, HH), jnp.float32),
        pltpu.SemaphoreType.DMA,
        pltpu.SemaphoreType.DMA,
    ],
    compiler_params=_SC_PARAMS,
)
def _agg_kernel(tab0, tab1, ei2, zrows_hbm, out_hbm,
                idx_v, rows_v, agg_sh, sem_i, sem_g):
    c = lax.axis_index("c")
    s = lax.axis_index("s")
    base = s * ROWS_PER_TILE

    def fire_idx(g, p):
        for i in range(AGRP):
            j = s + (g * AGRP + i) * NSUB
            pltpu.async_copy(ei2.at[j], idx_v.at[p, i], sem_i)

    def drain_idx(p):
        for i in range(AGRP):
            pltpu.make_async_copy(ei2.at[0], idx_v.at[p, i], sem_i).wait()

    def fire_gather(p):
        @pl.when(c == 0)
        def _():
            for i in range(AGRP):
                pltpu.async_copy(tab0.at[idx_v.at[p, i, 0]], rows_v.at[p, i],
                                 sem_g)

        @pl.when(c == 1)
        def _():
            for i in range(AGRP):
                pltpu.async_copy(tab1.at[idx_v.at[p, i, 0]], rows_v.at[p, i],
                                 sem_g)

    def drain_gather(p):
        for i in range(AGRP):
            pltpu.make_async_copy(tab0.at[idx_v.at[p, i, 0]], rows_v.at[p, i],
                                  sem_g).wait()

    def scatter(p):
        for i in range(AGRP):
            pltpu.sync_copy(rows_v.at[p, i], agg_sh.at[idx_v.at[p, i, 1]],
                            add=True)

    pltpu.sync_copy(zrows_hbm, agg_sh.at[pl.ds(base, ROWS_PER_TILE)])
    plsc.subcore_barrier()

    fire_idx(0, 0)
    drain_idx(0)
    fire_gather(0)
    fire_idx(1, 1)

    def body(g, carry):
        p = lax.rem(g, 2)
        q = 1 - p
        drain_gather(p)

        @pl.when(g + 1 < _ANG)
        def _():
            drain_idx(q)
            fire_gather(q)

        scatter(p)

        @pl.when(g + 2 < _ANG)
        def _():
            fire_idx(g + 2, p)

        return carry

    lax.fori_loop(0, _ANG, body, 0)
    plsc.subcore_barrier()
    pltpu.sync_copy(
        agg_sh.at[pl.ds(base, ROWS_PER_TILE)],
        out_hbm.at[c].at[pl.ds(base, ROWS_PER_TILE)],
    )


def _aggregate(table, ei2):
    zrows = jnp.zeros((ROWS_PER_TILE, HH), jnp.float32)
    return _agg_kernel(table[0], table[1], ei2, zrows)


_RB = 2000
_GRID = N // _RB


def _proj_body(x_ref, w_ref, b_ref, out_ref):
    y = jnp.dot(x_ref[...], w_ref[...], preferred_element_type=jnp.float32)
    y = y + b_ref[...]
    out_ref[0] = y[:, :HH]
    out_ref[1] = y[:, HH:]


def _project(x, W, b):
    return pl.pallas_call(
        _proj_body,
        grid=(_GRID,),
        in_specs=[
            pl.BlockSpec((_RB, D), lambda i: (i, 0)),
            pl.BlockSpec((D, H), lambda i: (0, 0)),
            pl.BlockSpec((1, H), lambda i: (0, 0)),
        ],
        out_specs=pl.BlockSpec((NCORE, _RB, HH), lambda i: (0, i, 0)),
        out_shape=jax.ShapeDtypeStruct((NCORE, N, HH), jnp.float32),
    )(x, W, b.reshape(1, H))


def _sage_mix(agg_ref, cnt_ref, xd_ref, wl_ref, bl_ref, wr_ref):
    cnt = jnp.maximum(cnt_ref[0] + cnt_ref[1], 1.0)
    mean = jnp.concatenate([agg_ref[0], agg_ref[1]], axis=1) / cnt
    xd = jnp.concatenate([xd_ref[0], xd_ref[1]], axis=1)
    y = (
        jnp.dot(mean, wl_ref[...], preferred_element_type=jnp.float32)
        + bl_ref[...]
        + jnp.dot(xd, wr_ref[...], preferred_element_type=jnp.float32)
    )
    return jnp.maximum(y, 0.0)


def _mid_body(agg_ref, cnt_ref, xd_ref, wl_ref, bl_ref, wr_ref, out_ref):
    y = _sage_mix(agg_ref, cnt_ref, xd_ref, wl_ref, bl_ref, wr_ref)
    out_ref[0] = y[:, :HH]
    out_ref[1] = y[:, HH:]


def _final_body(agg_ref, cnt_ref, xd_ref, wl_ref, bl_ref, wr_ref, out_ref):
    out_ref[...] = _sage_mix(agg_ref, cnt_ref, xd_ref, wl_ref, bl_ref, wr_ref)


def _head_body(agg_ref, cnt_ref, xd_ref, wl_ref, bl_ref, wr_ref,
               w1_ref, b1_ref, w2_ref, b2_ref, out_ref, logits_ref):
    y = _sage_mix(agg_ref, cnt_ref, xd_ref, wl_ref, bl_ref, wr_ref)
    out_ref[...] = y
    h = jnp.maximum(
        jnp.dot(y, w1_ref[...], preferred_element_type=jnp.float32) + b1_ref[...],
        0.0,
    )
    logits_ref[...] = (
        jnp.dot(h, w2_ref[...], preferred_element_type=jnp.float32) + b2_ref[...]
    )


_SAGE_SPECS = [
    pl.BlockSpec((NCORE, _RB, HH), lambda i: (0, i, 0)),
    pl.BlockSpec((NCORE, _RB, 1), lambda i: (0, i, 0)),
    pl.BlockSpec((NCORE, _RB, HH), lambda i: (0, i, 0)),
    pl.BlockSpec((H, H), lambda i: (0, 0)),
    pl.BlockSpec((1, H), lambda i: (0, 0)),
    pl.BlockSpec((H, H), lambda i: (0, 0)),
]


def _sage_mid(agg, cnt, xd, Wl, bl, Wr):
    return pl.pallas_call(
        _mid_body,
        grid=(_GRID,),
        in_specs=_SAGE_SPECS,
        out_specs=pl.BlockSpec((NCORE, _RB, HH), lambda i: (0, i, 0)),
        out_shape=jax.ShapeDtypeStruct((NCORE, N, HH), jnp.float32),
    )(agg, cnt, xd, Wl, bl.reshape(1, H), Wr)


def _sage_final(agg, cnt, xd, Wl, bl, Wr):
    return pl.pallas_call(
        _final_body,
        grid=(_GRID,),
        in_specs=_SAGE_SPECS,
        out_specs=pl.BlockSpec((_RB, H), lambda i: (i, 0)),
        out_shape=jax.ShapeDtypeStruct((N, H), jnp.float32),
    )(agg, cnt, xd, Wl, bl.reshape(1, H), Wr)


def _sage_head(agg, cnt, xd, Wl, bl, Wr, W1, b1, W2, b2):
    return pl.pallas_call(
        _head_body,
        grid=(_GRID,),
        in_specs=_SAGE_SPECS + [
            pl.BlockSpec((H, H), lambda i: (0, 0)),
            pl.BlockSpec((1, H), lambda i: (0, 0)),
            pl.BlockSpec((H, OUT), lambda i: (0, 0)),
            pl.BlockSpec((1, OUT), lambda i: (0, 0)),
        ],
        out_specs=[
            pl.BlockSpec((_RB, H), lambda i: (i, 0)),
            pl.BlockSpec((_RB, OUT), lambda i: (i, 0)),
        ],
        out_shape=[
            jax.ShapeDtypeStruct((N, H), jnp.float32),
            jax.ShapeDtypeStruct((N, OUT), jnp.float32),
        ],
    )(agg, cnt, xd, Wl, bl.reshape(1, H), Wr,
      W1, b1.reshape(1, H), W2, b2.reshape(1, OUT))


def kernel(x_transaction, x_account, edge_index_t2a, edge_index_a2t,
           W_in_t, b_in_t, W_in_a, b_in_a,
           c1_t2a_Wl, c1_t2a_bl, c1_t2a_Wr,
           c1_a2t_Wl, c1_a2t_bl, c1_a2t_Wr,
           c2_t2a_Wl, c2_t2a_bl, c2_t2a_Wr,
           c2_a2t_Wl, c2_a2t_bl, c2_a2t_Wr,
           head_W1, head_b1, head_W2, head_b2):
    pad_rows = NROWS_PAD - NROWS
    src_pad = jnp.zeros((pad_rows, 1, CHUNK), jnp.int32)
    dst_pad = jnp.full((pad_rows, 1, CHUNK), N, jnp.int32)

    def _edges(ei):
        e32 = ei.astype(jnp.int32).reshape(2, NROWS, CHUNK)
        src = jnp.concatenate([e32[0][:, None], src_pad])
        dst = jnp.concatenate([e32[1][:, None], dst_pad])
        return jnp.concatenate([src, dst], axis=1)

    ei_t2a = _edges(edge_index_t2a)
    ei_a2t = _edges(edge_index_a2t)

    cnt_t2a, cnt_a2t = _counts(ei_t2a, ei_a2t)

    t_tab = _project(x_transaction, W_in_t, b_in_t)
    a_tab = _project(x_account, W_in_a, b_in_a)

    agg_a = _aggregate(t_tab, ei_t2a)
    agg_t = _aggregate(a_tab, ei_a2t)
    xa1 = _sage_mid(agg_a, cnt_t2a, a_tab, c1_t2a_Wl, c1_t2a_bl, c1_t2a_Wr)
    xt1 = _sage_mid(agg_t, cnt_a2t, t_tab, c1_a2t_Wl, c1_a2t_bl, c1_a2t_Wr)

    agg_a2 = _aggregate(xt1, ei_t2a)
    agg_t2 = _aggregate(xa1, ei_a2t)
    xa2 = _sage_final(agg_a2, cnt_t2a, xa1, c2_t2a_Wl, c2_t2a_bl, c2_t2a_Wr)
    xt2, logits = _sage_head(agg_t2, cnt_a2t, xt1,
                             c2_a2t_Wl, c2_a2t_bl, c2_a2t_Wr,
                             head_W1, head_b1, head_W2, head_b2)
    return (logits, xt2, xa2)


# speedup vs baseline: 1.3692x; 1.0006x over previous
import functools

import jax
import jax.numpy as jnp
from jax import lax
from jax.experimental import pallas as pl
from jax.experimental.pallas import tpu as pltpu
from jax.experimental.pallas import tpu_sc as plsc

N = 50000
E = 800000
D = 128
H = 64
HH = 32
OUT = 2

CHUNK = 128
NROWS = E // CHUNK
AGRP = 3
CGRP = 6
NROWS_PAD = 6336
NSUB = 16
NCORE = 2
ROWS_PER_TILE = 3128
N_PAD = ROWS_PER_TILE * NSUB
SPAN = 3200
CNT_PAD = SPAN * NSUB

_mesh = plsc.VectorSubcoreMesh(core_axis_name="c", subcore_axis_name="s")
_SC_PARAMS = pltpu.CompilerParams(use_tc_tiling_on_sc=False)


_CW = 16
_CNG = NROWS_PAD // (NCORE * NSUB) // CGRP


@functools.partial(
    pl.kernel,
    out_type=[jax.ShapeDtypeStruct((NCORE, CNT_PAD, _CW), jnp.float32),
              jax.ShapeDtypeStruct((NCORE, CNT_PAD, _CW), jnp.float32)],
    mesh=_mesh,
    scratch_types=[
        pltpu.VMEM((2, CGRP, CHUNK), jnp.int32),
        pltpu.VMEM((CHUNK, _CW), jnp.float32),
        pltpu.VMEM_SHARED((CNT_PAD, _CW), jnp.float32),
        pltpu.SemaphoreType.DMA,
    ],
    compiler_params=_SC_PARAMS,
)
def _count_kernel(ei_a_hbm, ei_b_hbm, ones_hbm, zcol_hbm, out_a_hbm, out_b_hbm,
                  didx_v, ones_v, cnt_sh, sem):
    c = lax.axis_index("c")
    s = lax.axis_index("s")
    wid = c * NSUB + s
    stride = NCORE * NSUB
    pltpu.sync_copy(ones_hbm, ones_v)

    def one_pass(ei_hbm, out_hbm):
        pltpu.sync_copy(zcol_hbm, cnt_sh.at[pl.ds(s * SPAN, SPAN)])
        plsc.subcore_barrier()

        def fire_idx(g, p):
            for i in range(CGRP):
                pltpu.async_copy(ei_hbm.at[wid + (g * CGRP + i) * stride, 1],
                                 didx_v.at[p, i], sem)

        def drain_idx(p):
            for i in range(CGRP):
                pltpu.make_async_copy(ei_hbm.at[0, 1], didx_v.at[p, i],
                                      sem).wait()

        fire_idx(0, 0)

        def body(g, carry):
            p = lax.rem(g, 2)
            drain_idx(p)

            @pl.when(g + 1 < _CNG)
            def _():
                fire_idx(g + 1, 1 - p)

            for i in range(CGRP):
                pltpu.sync_copy(ones_v, cnt_sh.at[didx_v.at[p, i]], add=True)
            return carry

        lax.fori_loop(0, _CNG, body, 0)
        plsc.subcore_barrier()
        pltpu.sync_copy(
            cnt_sh.at[pl.ds(s * SPAN, SPAN)],
            out_hbm.at[c].at[pl.ds(s * SPAN, SPAN)],
        )

    one_pass(ei_a_hbm, out_a_hbm)
    one_pass(ei_b_hbm, out_b_hbm)


def _counts(ei_a, ei_b):
    ones = jnp.ones((CHUNK, _CW), jnp.float32)
    zcol = jnp.zeros((SPAN, _CW), jnp.float32)
    ca, cb = _count_kernel(ei_a, ei_b, ones, zcol)
    return ca[:, :, :1], cb[:, :, :1]


_ANG = NROWS_PAD // NSUB // AGRP


@functools.partial(
    pl.kernel,
    out_type=jax.ShapeDtypeStruct((NCORE, N_PAD, HH), jnp.float32),
    mesh=_mesh,
    scratch_types=[
        pltpu.VMEM((2, AGRP, 2, CHUNK), jnp.int32),
        pltpu.VMEM((2, AGRP, CHUNK, HH), jnp.float32),
        pltpu.VMEM_SHARED((N_PAD, HH), jnp.float32),
        pltpu.SemaphoreType.DMA,
        pltpu.SemaphoreType.DMA,
    ],
    compiler_params=_SC_PARAMS,
)
def _agg_kernel(tab0, tab1, ei2, zrows_hbm, out_hbm,
                idx_v, rows_v, agg_sh, sem_i, sem_g):
    c = lax.axis_index("c")
    s = lax.axis_index("s")
    base = s * ROWS_PER_TILE

    def fire_idx(g, p):
        for i in range(AGRP):
            j = s + (g * AGRP + i) * NSUB
            pltpu.async_copy(ei2.at[j], idx_v.at[p, i], sem_i)

    def drain_idx(p):
        for i in range(AGRP):
            pltpu.make_async_copy(ei2.at[0], idx_v.at[p, i], sem_i).wait()

    def fire_gather(p):
        @pl.when(c == 0)
        def _():
            for i in range(AGRP):
                pltpu.async_copy(tab0.at[idx_v.at[p, i, 0]], rows_v.at[p, i],
                                 sem_g)

        @pl.when(c == 1)
        def _():
            for i in range(AGRP):
                pltpu.async_copy(tab1.at[idx_v.at[p, i, 0]], rows_v.at[p, i],
                                 sem_g)

    def drain_gather(p):
        for i in range(AGRP):
            pltpu.make_async_copy(tab0.at[idx_v.at[p, i, 0]], rows_v.at[p, i],
                                  sem_g).wait()

    def scatter(p):
        for i in range(AGRP):
            pltpu.sync_copy(rows_v.at[p, i], agg_sh.at[idx_v.at[p, i, 1]],
                            add=True)

    fire_idx(0, 0)
    fire_idx(1, 1)
    pltpu.sync_copy(zrows_hbm, agg_sh.at[pl.ds(base, ROWS_PER_TILE)])
    drain_idx(0)
    fire_gather(0)
    plsc.subcore_barrier()

    def body(g, carry):
        p = lax.rem(g, 2)
        q = 1 - p
        drain_gather(p)

        @pl.when(g + 1 < _ANG)
        def _():
            drain_idx(q)
            fire_gather(q)

        scatter(p)

        @pl.when(g + 2 < _ANG)
        def _():
            fire_idx(g + 2, p)

        return carry

    lax.fori_loop(0, _ANG, body, 0)
    plsc.subcore_barrier()
    pltpu.sync_copy(
        agg_sh.at[pl.ds(base, ROWS_PER_TILE)],
        out_hbm.at[c].at[pl.ds(base, ROWS_PER_TILE)],
    )


def _aggregate(table, ei2):
    zrows = jnp.zeros((ROWS_PER_TILE, HH), jnp.float32)
    return _agg_kernel(table[0], table[1], ei2, zrows)


_RB = 2000
_GRID = N // _RB


def _proj_body(x_ref, w_ref, b_ref, out_ref):
    y = jnp.dot(x_ref[...], w_ref[...], preferred_element_type=jnp.float32)
    y = y + b_ref[...]
    out_ref[0] = y[:, :HH]
    out_ref[1] = y[:, HH:]


def _project(x, W, b):
    return pl.pallas_call(
        _proj_body,
        grid=(_GRID,),
        in_specs=[
            pl.BlockSpec((_RB, D), lambda i: (i, 0)),
            pl.BlockSpec((D, H), lambda i: (0, 0)),
            pl.BlockSpec((1, H), lambda i: (0, 0)),
        ],
        out_specs=pl.BlockSpec((NCORE, _RB, HH), lambda i: (0, i, 0)),
        out_shape=jax.ShapeDtypeStruct((NCORE, N, HH), jnp.float32),
    )(x, W, b.reshape(1, H))


def _sage_mix(agg_ref, cnt_ref, xd_ref, wl_ref, bl_ref, wr_ref):
    cnt = jnp.maximum(cnt_ref[0] + cnt_ref[1], 1.0)
    mean = jnp.concatenate([agg_ref[0], agg_ref[1]], axis=1) / cnt
    xd = jnp.concatenate([xd_ref[0], xd_ref[1]], axis=1)
    y = (
        jnp.dot(mean, wl_ref[...], preferred_element_type=jnp.float32)
        + bl_ref[...]
        + jnp.dot(xd, wr_ref[...], preferred_element_type=jnp.float32)
    )
    return jnp.maximum(y, 0.0)


def _mid_body(agg_ref, cnt_ref, xd_ref, wl_ref, bl_ref, wr_ref, out_ref):
    y = _sage_mix(agg_ref, cnt_ref, xd_ref, wl_ref, bl_ref, wr_ref)
    out_ref[0] = y[:, :HH]
    out_ref[1] = y[:, HH:]


def _final_body(agg_ref, cnt_ref, xd_ref, wl_ref, bl_ref, wr_ref, out_ref):
    out_ref[...] = _sage_mix(agg_ref, cnt_ref, xd_ref, wl_ref, bl_ref, wr_ref)


def _head_body(agg_ref, cnt_ref, xd_ref, wl_ref, bl_ref, wr_ref,
               w1_ref, b1_ref, w2_ref, b2_ref, out_ref, logits_ref):
    y = _sage_mix(agg_ref, cnt_ref, xd_ref, wl_ref, bl_ref, wr_ref)
    out_ref[...] = y
    h = jnp.maximum(
        jnp.dot(y, w1_ref[...], preferred_element_type=jnp.float32) + b1_ref[...],
        0.0,
    )
    logits_ref[...] = (
        jnp.dot(h, w2_ref[...], preferred_element_type=jnp.float32) + b2_ref[...]
    )


_SAGE_SPECS = [
    pl.BlockSpec((NCORE, _RB, HH), lambda i: (0, i, 0)),
    pl.BlockSpec((NCORE, _RB, 1), lambda i: (0, i, 0)),
    pl.BlockSpec((NCORE, _RB, HH), lambda i: (0, i, 0)),
    pl.BlockSpec((H, H), lambda i: (0, 0)),
    pl.BlockSpec((1, H), lambda i: (0, 0)),
    pl.BlockSpec((H, H), lambda i: (0, 0)),
]


def _sage_mid(agg, cnt, xd, Wl, bl, Wr):
    return pl.pallas_call(
        _mid_body,
        grid=(_GRID,),
        in_specs=_SAGE_SPECS,
        out_specs=pl.BlockSpec((NCORE, _RB, HH), lambda i: (0, i, 0)),
        out_shape=jax.ShapeDtypeStruct((NCORE, N, HH), jnp.float32),
    )(agg, cnt, xd, Wl, bl.reshape(1, H), Wr)


def _sage_final(agg, cnt, xd, Wl, bl, Wr):
    return pl.pallas_call(
        _final_body,
        grid=(_GRID,),
        in_specs=_SAGE_SPECS,
        out_specs=pl.BlockSpec((_RB, H), lambda i: (i, 0)),
        out_shape=jax.ShapeDtypeStruct((N, H), jnp.float32),
    )(agg, cnt, xd, Wl, bl.reshape(1, H), Wr)


def _sage_head(agg, cnt, xd, Wl, bl, Wr, W1, b1, W2, b2):
    return pl.pallas_call(
        _head_body,
        grid=(_GRID,),
        in_specs=_SAGE_SPECS + [
            pl.BlockSpec((H, H), lambda i: (0, 0)),
            pl.BlockSpec((1, H), lambda i: (0, 0)),
            pl.BlockSpec((H, OUT), lambda i: (0, 0)),
            pl.BlockSpec((1, OUT), lambda i: (0, 0)),
        ],
        out_specs=[
            pl.BlockSpec((_RB, H), lambda i: (i, 0)),
            pl.BlockSpec((_RB, OUT), lambda i: (i, 0)),
        ],
        out_shape=[
            jax.ShapeDtypeStruct((N, H), jnp.float32),
            jax.ShapeDtypeStruct((N, OUT), jnp.float32),
        ],
    )(agg, cnt, xd, Wl, bl.reshape(1, H), Wr,
      W1, b1.reshape(1, H), W2, b2.reshape(1, OUT))


def kernel(x_transaction, x_account, edge_index_t2a, edge_index_a2t,
           W_in_t, b_in_t, W_in_a, b_in_a,
           c1_t2a_Wl, c1_t2a_bl, c1_t2a_Wr,
           c1_a2t_Wl, c1_a2t_bl, c1_a2t_Wr,
           c2_t2a_Wl, c2_t2a_bl, c2_t2a_Wr,
           c2_a2t_Wl, c2_a2t_bl, c2_a2t_Wr,
           head_W1, head_b1, head_W2, head_b2):
    pad_rows = NROWS_PAD - NROWS
    src_pad = jnp.zeros((pad_rows, 1, CHUNK), jnp.int32)
    dst_pad = jnp.full((pad_rows, 1, CHUNK), N, jnp.int32)

    def _edges(ei):
        e32 = ei.astype(jnp.int32).reshape(2, NROWS, CHUNK)
        src = jnp.concatenate([e32[0][:, None], src_pad])
        dst = jnp.concatenate([e32[1][:, None], dst_pad])
        return jnp.concatenate([src, dst], axis=1)

    ei_t2a = _edges(edge_index_t2a)
    ei_a2t = _edges(edge_index_a2t)

    cnt_t2a, cnt_a2t = _counts(ei_t2a, ei_a2t)

    t_tab = _project(x_transaction, W_in_t, b_in_t)
    a_tab = _project(x_account, W_in_a, b_in_a)

    agg_a = _aggregate(t_tab, ei_t2a)
    agg_t = _aggregate(a_tab, ei_a2t)
    xa1 = _sage_mid(agg_a, cnt_t2a, a_tab, c1_t2a_Wl, c1_t2a_bl, c1_t2a_Wr)
    xt1 = _sage_mid(agg_t, cnt_a2t, t_tab, c1_a2t_Wl, c1_a2t_bl, c1_a2t_Wr)

    agg_a2 = _aggregate(xt1, ei_t2a)
    agg_t2 = _aggregate(xa1, ei_a2t)
    xa2 = _sage_final(agg_a2, cnt_t2a, xa1, c2_t2a_Wl, c2_t2a_bl, c2_t2a_Wr)
    xt2, logits = _sage_head(agg_t2, cnt_a2t, xt1,
                             c2_a2t_Wl, c2_a2t_bl, c2_a2t_Wr,
                             head_W1, head_b1, head_W2, head_b2)
    return (logits, xt2, xa2)
